# baseline (speedup 1.0000x reference)
import functools

import jax
import jax.numpy as jnp
from jax import lax
from jax.experimental import pallas as pl
from jax.experimental.pallas import tpu as pltpu
from jax.experimental.pallas import tpu_sc as plsc

N = 4194304
NUM_TYPES = 64
NC = 2
NS = 16
L = 16
NW = NC * NS
PER_W = N // NW
CHUNK = 8192
ROWS = CHUNK // 128
NGROUP = PER_W // CHUNK
NBUFN = 4
SB = 3
RB = 5


def _sc_kernel(numbers_hbm, energy_hbm, table_hbm, out_hbm, tbl_v, esh, *scratch):
    idx_bufs = scratch[0:NBUFN]
    sh_bufs = scratch[NBUFN:NBUFN + SB]
    row_bufs = scratch[NBUFN + SB:NBUFN + SB + RB]
    nsems = scratch[NBUFN + SB + RB:2 * NBUFN + SB + RB]
    ssems = scratch[2 * NBUFN + SB + RB:2 * NBUFN + 2 * SB + RB]
    esems = scratch[2 * NBUFN + 2 * SB + RB:2 * NBUFN + 2 * SB + 2 * RB]
    osems = scratch[2 * NBUFN + 2 * SB + 2 * RB:2 * NBUFN + 2 * SB + 3 * RB]

    cid = lax.axis_index("c")
    sid = lax.axis_index("s")
    wid = sid * NC + cid
    wbase = wid * PER_W
    wrow = wid * (PER_W // 128)
    pltpu.sync_copy(table_hbm, tbl_v)

    for r in range(RB):
        base = r * NS * ROWS + sid * ROWS
        for k in range(ROWS // L):
            row_bufs[r][pl.ds(k * L, L)] = (
                lax.iota(jnp.int32, L) + (base + k * L)
            )

    nin, scat, ein, out = {}, {}, {}, {}

    def fire_nin(g):
        nin[g] = pltpu.async_copy(
            numbers_hbm.at[pl.ds(wbase + g * CHUNK, CHUNK)],
            idx_bufs[g % NBUFN], nsems[g % NBUFN])

    def fire_ein(g):
        r = g % RB
        ein[g] = pltpu.async_copy(
            energy_hbm.at[pl.ds(wrow + g * ROWS, ROWS)],
            esh.at[pl.ds(r * NS * ROWS + sid * ROWS, ROWS)], esems[r])

    def fire_out(g):
        r = g % RB
        out[g] = pltpu.async_copy(
            esh.at[pl.ds(r * NS * ROWS + sid * ROWS, ROWS)],
            out_hbm.at[pl.ds(wrow + g * ROWS, ROWS)], osems[r])

    for g in range(2):
        fire_ein(g)
    for g in range(3):
        fire_nin(g)

    for g in range(NGROUP):
        sb = g % SB
        nin.pop(g).wait()
        ein.pop(g).wait()

        idx_b = idx_bufs[g % NBUFN]
        sh_b = sh_bufs[sb]

        @plsc.parallel_loop(0, CHUNK // L, unroll=8)
        def _(i):
            sh = plsc.load_gather(tbl_v, [idx_b[pl.ds(i * L, L)]])
            sh_b[i // 8, pl.ds((i % 8) * L, L)] = sh

        scat[g] = pltpu.async_copy(
            sh_b, esh.at[row_bufs[g % RB]], ssems[sb], add=True)

        if g >= 1:
            scat.pop(g - 1).wait()
            fire_out(g - 1)
        if g + 2 < NGROUP:
            if g >= 3:
                out.pop(g - 3).wait()
            fire_ein(g + 2)
        if g + 3 < NGROUP:
            fire_nin(g + 3)

    scat.pop(NGROUP - 1).wait()
    fire_out(NGROUP - 1)
    for g in sorted(out):
        out[g].wait()


def kernel(numbers, energy, table):
    tbl_flat = table.reshape(NUM_TYPES)
    energy_2d = energy.reshape(N // 128, 128)
    mesh = plsc.VectorSubcoreMesh(core_axis_name="c", subcore_axis_name="s")
    run = functools.partial(
        pl.kernel,
        mesh=mesh,
        out_type=jax.ShapeDtypeStruct((N // 128, 128), jnp.float32),
        scratch_types=(
            [pltpu.VMEM((NUM_TYPES,), jnp.float32)]
            + [pltpu.VMEM_SHARED((RB * NS * ROWS, 128), jnp.float32)]
            + [pltpu.VMEM((CHUNK,), jnp.int32) for _ in range(NBUFN)]
            + [pltpu.VMEM((ROWS, 128), jnp.float32) for _ in range(SB)]
            + [pltpu.VMEM((ROWS,), jnp.int32) for _ in range(RB)]
            + [pltpu.SemaphoreType.DMA for _ in range(NBUFN + SB + 2 * RB)]
        ),
        compiler_params=pltpu.CompilerParams(needs_layout_passes=False),
    )(_sc_kernel)
    return run(numbers, energy_2d, tbl_flat).reshape(N)


# speedup vs baseline: 1.0144x; 1.0144x over previous
import functools

import jax
import jax.numpy as jnp
from jax import lax
from jax.experimental import pallas as pl
from jax.experimental.pallas import tpu as pltpu
from jax.experimental.pallas import tpu_sc as plsc

N = 4194304
NUM_TYPES = 64
NC = 2
NS = 16
L = 16
NW = NC * NS
PER_W = N // NW
CHUNK = 8192
NCHUNK = PER_W // CHUNK
NBUF = 6


def _sc_kernel(numbers_hbm, energy_hbm, table_hbm, out_hbm, tbl_v, *scratch):
    idx_bufs = scratch[0:NBUF]
    en_bufs = scratch[NBUF:2 * NBUF]
    nsems = scratch[2 * NBUF:3 * NBUF]
    esems = scratch[3 * NBUF:4 * NBUF]
    osems = scratch[4 * NBUF:5 * NBUF]

    wid = lax.axis_index("s") * NC + lax.axis_index("c")
    wbase = wid * PER_W
    pltpu.sync_copy(table_hbm, tbl_v)

    in_handles = {}
    out_handles = {}

    def start_in(c):
        b = c % NBUF
        s = pl.ds(wbase + c * CHUNK, CHUNK)
        hn = pltpu.async_copy(numbers_hbm.at[s], idx_bufs[b], nsems[b])
        he = pltpu.async_copy(energy_hbm.at[s], en_bufs[b], esems[b])
        in_handles[c] = (hn, he)

    def start_out(c):
        b = c % NBUF
        s = pl.ds(wbase + c * CHUNK, CHUNK)
        out_handles[c] = pltpu.async_copy(en_bufs[b], out_hbm.at[s], osems[b])

    for c in range(NBUF - 1):
        start_in(c)

    for c in range(NCHUNK):
        b = c % NBUF
        for h in in_handles.pop(c):
            h.wait()

        idx_b = idx_bufs[b]
        en_b = en_bufs[b]

        @plsc.parallel_loop(0, CHUNK // L, unroll=8)
        def _(i):
            s = pl.ds(i * L, L)
            sh = plsc.load_gather(tbl_v, [idx_b[s]])
            plsc.addupdate(en_b.at[s], sh)

        start_out(c)
        n = c + NBUF - 1
        if n < NCHUNK:
            if c >= 1:
                out_handles.pop(c - 1).wait()
            start_in(n)

    for c in sorted(out_handles):
        out_handles[c].wait()


def kernel(numbers, energy, table):
    tbl_flat = table.reshape(NUM_TYPES)
    mesh = plsc.VectorSubcoreMesh(core_axis_name="c", subcore_axis_name="s")
    run = functools.partial(
        pl.kernel,
        mesh=mesh,
        out_type=jax.ShapeDtypeStruct((N,), jnp.float32),
        scratch_types=(
            [pltpu.VMEM((NUM_TYPES,), jnp.float32)]
            + [pltpu.VMEM((CHUNK,), jnp.int32) for _ in range(NBUF)]
            + [pltpu.VMEM((CHUNK,), jnp.float32) for _ in range(NBUF)]
            + [pltpu.SemaphoreType.DMA for _ in range(3 * NBUF)]
        ),
        compiler_params=pltpu.CompilerParams(needs_layout_passes=False),
    )(_sc_kernel)
    return run(numbers, energy, tbl_flat)
